# Initial kernel scaffold; baseline (speedup 1.0000x reference)
import jax
import jax.numpy as jnp
from jax.experimental import pallas as pl


def kernel(xyz, features, W0, b0, W1, b1, W2, b2):
    raise NotImplementedError("write your pallas kernel here")


# speedup vs baseline: 25.4592x; 25.4592x over previous
import functools

import numpy as np
import jax
import jax.numpy as jnp
from jax import lax
from jax.experimental import pallas as pl
from jax.experimental.pallas import tpu as pltpu
from jax.experimental.pallas import tpu_sc as plsc

_B, _N, _C = 8, 8192, 16
_S, _NS = 1024, 32
_R2 = np.float32(np.float64(0.2) ** 2)
_D = 24
_DT = 128
_NW = 32
_CPT = _B * _S // _NW
_RPT = _CPT * _NS
_CHUNK = 512
_K = 16


def _fps_body(xt_ref, nxt_ref, dists_ref):
    x = xt_ref[0]
    y = xt_ref[1]
    z = xt_ref[2]
    lane = lax.broadcasted_iota(jnp.int32, (_B, _N), 1)
    lane_s = lax.broadcasted_iota(jnp.int32, (_B, _S), 1)
    nxt_ref[...] = jnp.zeros((3, _B, _S), jnp.float32)
    dists_ref[...] = jnp.full((_B, _N), 1e10, jnp.float32)

    def body(i, far):
        sel = lane == far
        cx = jnp.max(jnp.where(sel, x, -1.0), axis=1, keepdims=True)
        cy = jnp.max(jnp.where(sel, y, -1.0), axis=1, keepdims=True)
        cz = jnp.max(jnp.where(sel, z, -1.0), axis=1, keepdims=True)
        hit = lane_s == i
        nxt_ref[0] = jnp.where(hit, cx, nxt_ref[0])
        nxt_ref[1] = jnp.where(hit, cy, nxt_ref[1])
        nxt_ref[2] = jnp.where(hit, cz, nxt_ref[2])
        dx = x - cx
        dy = y - cy
        dz = z - cz
        d = dx * dx + dy * dy + dz * dz
        dn = jnp.minimum(dists_ref[...], d)
        dists_ref[...] = dn
        m = jnp.max(dn, axis=1, keepdims=True)
        return jnp.min(jnp.where(dn == m, lane, _N), axis=1, keepdims=True)

    lax.fori_loop(0, _S, body, jnp.zeros((_B, 1), jnp.int32))


_fps = pl.pallas_call(
    _fps_body,
    out_shape=jax.ShapeDtypeStruct((3, _B, _S), jnp.float32),
    scratch_shapes=[pltpu.VMEM((_B, _N), jnp.float32)],
)


def _bq_body(xt_hbm, nxt_hbm, out_hbm, xv, cv, ov):
    wid = lax.axis_index("s") * 2 + lax.axis_index("c")
    b = wid // 4
    q = wid - b * 4
    pltpu.sync_copy(xt_hbm.at[:, b, :], xv)
    pltpu.sync_copy(nxt_hbm.at[:, b, pl.ds(q * _CPT, _CPT)], cv)
    iota16 = lax.iota(jnp.int32, 16)
    gbase = b * _N

    def group(g, carry):
        c0 = g * 16
        cx = cv[0, pl.ds(c0, 16)]
        cy = cv[1, pl.ds(c0, 16)]
        cz = cv[2, pl.ds(c0, 16)]
        obase = (c0 + iota16) * _NS

        def scan_body(blk, cnt):
            p = blk * _K
            px16 = xv[0, pl.ds(p, _K)]
            py16 = xv[1, pl.ds(p, _K)]
            pz16 = xv[2, pl.ds(p, _K)]
            for k in range(_K):
                dx = jnp.full((16,), px16[k]) - cx
                dy = jnp.full((16,), py16[k]) - cy
                dz = jnp.full((16,), pz16[k]) - cz
                dd = dx * dx + dy * dy + dz * dz
                hit = jnp.logical_and(dd < _R2, cnt < _NS)
                plsc.store_scatter(ov, [obase + cnt],
                                   jnp.full((16,), gbase + p + k, jnp.int32),
                                   mask=hit)
                cnt = cnt + hit.astype(jnp.int32)
            return cnt

        cnt = lax.fori_loop(0, _N // _K, scan_body,
                            jnp.zeros((16,), jnp.int32))
        first = plsc.load_gather(ov, [obase])
        for j in range(1, _NS):
            plsc.store_scatter(ov, [obase + j], first, mask=cnt <= j)
        return carry

    lax.fori_loop(0, _CPT // 16, group, 0)
    pltpu.sync_copy(ov, out_hbm.at[pl.ds(wid * _RPT, _RPT)])


def _gather_body(tab_hbm, idx_hbm, out_hbm, idx_v, rows_v, sem):
    wid = lax.axis_index("s") * 2 + lax.axis_index("c")
    rbase = wid * (_RPT // 128)

    def chunk(c, carry):
        off = rbase + c * (_CHUNK // 128)
        pltpu.sync_copy(idx_hbm.at[pl.ds(off, _CHUNK // 128)], idx_v)
        copies = [
            pltpu.async_copy(tab_hbm.at[idx_v.at[j]], rows_v.at[j], sem)
            for j in range(_CHUNK // 128)
        ]
        for cp in copies:
            cp.wait()
        pltpu.sync_copy(rows_v, out_hbm.at[pl.ds(off, _CHUNK // 128)])
        return carry

    lax.fori_loop(0, _RPT // _CHUNK, chunk, 0)


@functools.cache
def _sc_kernels():
    mesh = plsc.VectorSubcoreMesh(core_axis_name="c", subcore_axis_name="s")
    bq = functools.partial(
        pl.kernel,
        mesh=mesh,
        compiler_params=pltpu.CompilerParams(needs_layout_passes=False),
        out_type=jax.ShapeDtypeStruct((_B * _S * _NS,), jnp.int32),
        scratch_types=[
            pltpu.VMEM((3, _N), jnp.float32),
            pltpu.VMEM((3, _CPT), jnp.float32),
            pltpu.VMEM((_RPT,), jnp.int32),
        ],
    )(_bq_body)
    gather = functools.partial(
        pl.kernel,
        mesh=mesh,
        compiler_params=pltpu.CompilerParams(needs_layout_passes=False),
        out_type=jax.ShapeDtypeStruct((_B * _S * _NS // 128, 128, _DT),
                                      jnp.float32),
        scratch_types=[
            pltpu.VMEM((_CHUNK // 128, 128), jnp.int32),
            pltpu.VMEM((_CHUNK // 128, 128, _DT), jnp.float32),
            pltpu.SemaphoreType.DMA,
        ],
    )(_gather_body)
    return bq, gather


def _dot_t(a, w):
    return lax.dot_general(a, w, (((1,), (1,)), ((), ())),
                           preferred_element_type=jnp.float32,
                           precision=lax.Precision.HIGHEST)


def _mlp_body(g_ref, nxt_ref, w0_ref, b0_ref, w1_ref, b1_ref, w2_ref, b2_ref,
              feat_ref, nxout_ref):
    blk = _S // 4
    x = g_ref[0, 0]
    h = _dot_t(x, w0_ref[...])
    nxt = nxt_ref[0]
    off = lax.dot_general(nxt, w0_ref[:, :3], (((0,), (1,)), ((), ())),
                          preferred_element_type=jnp.float32,
                          precision=lax.Precision.HIGHEST)
    offe = jnp.reshape(jnp.broadcast_to(off[:, None, :], (blk, _NS, 32)),
                       (blk * _NS, 32))
    h = jnp.maximum(h + b0_ref[...] - offe, 0.0)
    h = jnp.maximum(_dot_t(h, w1_ref[...]) + b1_ref[...], 0.0)
    h = jnp.maximum(_dot_t(h, w2_ref[...]) + b2_ref[...], 0.0)
    m = jnp.max(jnp.reshape(h, (blk, _NS, 64)), axis=1)
    feat_ref[0] = m.T
    nxout_ref[0] = nxt.T


_mlp_in_specs = [
    pl.BlockSpec((1, 1, _S // 4 * _NS, _DT), lambda b, q: (b, q, 0, 0)),
    pl.BlockSpec((1, 3, _S // 4), lambda b, q: (b, 0, q)),
    pl.BlockSpec((32, _DT), lambda b, q: (0, 0)),
    pl.BlockSpec((1, 32), lambda b, q: (0, 0)),
    pl.BlockSpec((32, 32), lambda b, q: (0, 0)),
    pl.BlockSpec((1, 32), lambda b, q: (0, 0)),
    pl.BlockSpec((64, 32), lambda b, q: (0, 0)),
    pl.BlockSpec((1, 64), lambda b, q: (0, 0)),
]
_mlp_out_specs = [
    pl.BlockSpec((1, 64, _S // 4), lambda b, q: (b, 0, q)),
    pl.BlockSpec((1, _S // 4, 3), lambda b, q: (b, q, 0)),
]
_mlp = pl.pallas_call(
    _mlp_body,
    grid=(_B, 4),
    in_specs=_mlp_in_specs,
    out_specs=_mlp_out_specs,
    out_shape=[
        jax.ShapeDtypeStruct((_B, 64, _S), jnp.float32),
        jax.ShapeDtypeStruct((_B, _S, 3), jnp.float32),
    ],
)


def kernel(xyz, features, W0, b0, W1, b1, W2, b2):
    bq, gather = _sc_kernels()
    xt = jnp.transpose(xyz, (2, 0, 1))
    nxt = _fps(xt)
    idx = bq(xt, nxt)
    feats_t = jnp.transpose(features, (0, 2, 1))
    tab = jnp.concatenate(
        [xyz, feats_t, jnp.zeros((_B, _N, _DT - 3 - _C), jnp.float32)], axis=-1
    ).reshape(_B * _N, _DT)
    g = gather(tab, idx.reshape(_B * _S * _NS // 128, 128))
    g4 = g.reshape(_B, 4, _S // 4 * _NS, _DT)
    w0e = jnp.concatenate([W0, jnp.zeros((32, _DT - _C - 3), jnp.float32)], axis=1)
    feat, nxout = _mlp(g4, jnp.transpose(nxt, (1, 0, 2)), w0e, b0.reshape(1, 32),
                       W1, b1.reshape(1, 32), W2, b2.reshape(1, 64))
    return nxout, feat
